# baseline (speedup 1.0000x reference)
import functools

import jax
import jax.numpy as jnp
import numpy as np
from jax import lax
from jax.experimental import pallas as pl
from jax.experimental.pallas import tpu as pltpu
from jax.experimental.pallas import tpu_sc as plsc

N = 10000
E = 320000
D = 128
B = 100
NODES = 100

NC = 2
NS = 16
NW = NC * NS
CH = 128
T_FAST = 160
E_PAD = NS * T_FAST * CH
N_PAD = 10240
RPT = N_PAD // NS

_ZROW = np.zeros((N_PAD, D), np.float32)
_ZDEG = np.zeros((N_PAD,), np.float32)

_mesh = plsc.VectorSubcoreMesh(core_axis_name="c", subcore_axis_name="s")


def _sc_body(with_deg, x_hbm, src_hbm, dst_hbm, zrow_hbm, zdeg_hbm,
             agg_hbm, deg_hbm,
             acc, rows0, rows1, sb0, sb1, sb2, sb3, db0, db1, hist,
             semg0, semg1, semd0, semd1, ss0, ss1, ss2, ss3, semsc):
    c = lax.axis_index("c")
    s = lax.axis_index("s")
    nch = jnp.where(c == 0, T_FAST, 0)
    ebase = s * (T_FAST * CH)

    @pl.when(c == 0)
    def _():
        pltpu.sync_copy(zrow_hbm.at[pl.ds(s * RPT, RPT)],
                        acc.at[pl.ds(s * RPT, RPT)])
        if with_deg:
            pltpu.sync_copy(zdeg_hbm, hist)
    plsc.subcore_barrier()

    rows = (rows0, rows1)
    semg = (semg0, semg1)
    dstb = (db0, db1)
    semd = (semd0, semd1)
    srcb = (sb0, sb1, sb2, sb3)
    sems = (ss0, ss1, ss2, ss3)

    def srcload(g, j):
        return pltpu.make_async_copy(
            src_hbm.at[pl.ds(ebase + g * CH, CH)], srcb[j], sems[j])

    def dstload(g, j):
        return pltpu.make_async_copy(
            dst_hbm.at[pl.ds(ebase + g * CH, CH)], dstb[j], semd[j])

    def gather(j4, j2):
        return pltpu.make_async_copy(x_hbm.at[srcb[j4]], rows[j2], semg[j2])

    @pl.when(c == 0)
    def _():
        for j in range(4):
            srcload(j, j).start()
        for b in range(2):
            srcload(b, b).wait()
            gather(b, b).start()
            dstload(b, b).start()

    def step(k4, carry):
        for b in range(4):
            g = k4 * 4 + b
            j2 = b % 2
            gather(b, j2).wait()
            dstload(g, j2).wait()
            db = dstb[j2]
            sc = pltpu.async_copy(rows[j2], acc.at[db], semsc, add=True)
            if with_deg:
                for i in range(CH // 16):
                    dv = db[pl.ds(i * 16, 16)]
                    cnt, last = plsc.scan_count(dv)
                    old = plsc.load_gather(hist, [dv])
                    plsc.store_scatter(hist, [dv], old + cnt.astype(jnp.float32),
                                       mask=last)
            sc.wait()

            @pl.when(g + 4 < nch)
            def _():
                srcload(g + 4, b).start()

            @pl.when(g + 2 < nch)
            def _():
                srcload(g + 2, (b + 2) % 4).wait()
                gather((b + 2) % 4, j2).start()
                dstload(g + 2, j2).start()
        return carry

    lax.fori_loop(0, nch // 4, step, 0)
    plsc.subcore_barrier()

    @pl.when(c == 0)
    def _():
        if with_deg:
            pltpu.sync_copy(hist, deg_hbm.at[pl.ds(s * N_PAD, N_PAD)])
        pltpu.sync_copy(acc.at[pl.ds(s * RPT, RPT)],
                        agg_hbm.at[pl.ds(s * RPT, RPT)])


def _make_sc(with_deg):
    return functools.partial(
        pl.kernel,
        out_type=[
            jax.ShapeDtypeStruct((N_PAD, D), jnp.float32),
            jax.ShapeDtypeStruct((NS * N_PAD,), jnp.float32),
        ],
        mesh=_mesh,
        compiler_params=pltpu.CompilerParams(needs_layout_passes=False),
        scratch_types=[
            pltpu.VMEM_SHARED((N_PAD, D), jnp.float32),
            pltpu.VMEM((CH, D), jnp.float32),
            pltpu.VMEM((CH, D), jnp.float32),
            pltpu.VMEM((CH,), jnp.int32),
            pltpu.VMEM((CH,), jnp.int32),
            pltpu.VMEM((CH,), jnp.int32),
            pltpu.VMEM((CH,), jnp.int32),
            pltpu.VMEM((CH,), jnp.int32),
            pltpu.VMEM((CH,), jnp.int32),
            pltpu.VMEM((N_PAD,), jnp.float32),
            pltpu.SemaphoreType.DMA,
            pltpu.SemaphoreType.DMA,
            pltpu.SemaphoreType.DMA,
            pltpu.SemaphoreType.DMA,
            pltpu.SemaphoreType.DMA,
            pltpu.SemaphoreType.DMA,
            pltpu.SemaphoreType.DMA,
            pltpu.SemaphoreType.DMA,
            pltpu.SemaphoreType.DMA,
        ],
    )(functools.partial(_sc_body, with_deg))


_sc_conv1 = _make_sc(True)
_sc_conv2 = _make_sc(False)


def _tc_conv_body(agg_ref, deg_ref, W_ref, b_ref, g_ref, be_ref, out_ref):
    a = agg_ref[:N, :]
    dg = jnp.sum(deg_ref[...], axis=1, keepdims=True)[:N, :]
    a = a * (1.0 / jnp.maximum(dg, 1.0))
    h = jnp.dot(a, W_ref[...], preferred_element_type=jnp.float32) + b_ref[...]
    m = jnp.mean(h, axis=0, keepdims=True)
    v = jnp.mean(h * h, axis=0, keepdims=True) - m * m
    hn = g_ref[...] * (h - m) * lax.rsqrt(v + 1e-5) + be_ref[...]
    out_ref[...] = jnp.maximum(hn, 0.0)


_tc_conv = pl.pallas_call(
    _tc_conv_body,
    out_shape=jax.ShapeDtypeStruct((N, D), jnp.float32),
    in_specs=[pl.BlockSpec(memory_space=pltpu.VMEM)] * 6,
    out_specs=pl.BlockSpec(memory_space=pltpu.VMEM),
)


def _tc_final_body(agg_ref, deg_ref, batch_ref,
                   W_ref, b_ref, g_ref, be_ref,
                   Ws1_ref, bs1_ref, Ws2_ref, bs2_ref,
                   Wh1_ref, bh1_ref, Wh2_ref, bh2_ref, Wh3_ref, bh3_ref,
                   out_ref):
    a = agg_ref[:N, :]
    dg = jnp.sum(deg_ref[...], axis=1, keepdims=True)[:N, :]
    a = a * (1.0 / jnp.maximum(dg, 1.0))
    h = jnp.dot(a, W_ref[...], preferred_element_type=jnp.float32) + b_ref[...]
    m = jnp.mean(h, axis=0, keepdims=True)
    v = jnp.mean(h * h, axis=0, keepdims=True) - m * m
    hn = g_ref[...] * (h - m) * lax.rsqrt(v + 1e-5) + be_ref[...]
    h2 = jnp.maximum(hn, 0.0)

    gids = lax.broadcasted_iota(jnp.int32, (B, N), 0)
    pmat = jnp.where(gids == batch_ref[...], 1.0 / NODES, 0.0)
    xg = jnp.dot(pmat, h2, preferred_element_type=jnp.float32)

    sh = jnp.maximum(xg, 0.0)
    sh = jnp.dot(sh, Ws1_ref[...], preferred_element_type=jnp.float32) + bs1_ref[...]
    sh = jnp.maximum(jnp.dot(sh, Ws2_ref[...], preferred_element_type=jnp.float32) + bs2_ref[...], 0.0)
    o = jnp.maximum(jnp.dot(sh, Wh1_ref[...], preferred_element_type=jnp.float32) + bh1_ref[...], 0.0)
    o = jnp.maximum(jnp.dot(o, Wh2_ref[...], preferred_element_type=jnp.float32) + bh2_ref[...], 0.0)
    out_ref[...] = jnp.dot(o, Wh3_ref[...], preferred_element_type=jnp.float32) + bh3_ref[...]


_tc_final = pl.pallas_call(
    _tc_final_body,
    out_shape=jax.ShapeDtypeStruct((B, 10), jnp.float32),
    in_specs=[pl.BlockSpec(memory_space=pltpu.VMEM)] * 17,
    out_specs=pl.BlockSpec(memory_space=pltpu.VMEM),
)


def kernel(x, edge_index, batch, W1, b1, g1, be1, W2, b2, g2, be2,
           Ws1, bs1, Ws2, bs2, Wh1, bh1, Wh2, bh2, Wh3, bh3):
    pad = E_PAD - E
    srcp = jnp.concatenate([edge_index[0], jnp.zeros((pad,), jnp.int32)])
    trash = N + (jnp.arange(pad, dtype=jnp.int32) % (N_PAD - N))
    dstp = jnp.concatenate([edge_index[1], trash])

    zrow = jnp.asarray(_ZROW)
    zdeg = jnp.asarray(_ZDEG)

    agg1, deg1 = _sc_conv1(x, srcp, dstp, zrow, zdeg)
    deg1 = deg1.reshape(NS, N_PAD).T
    h1 = _tc_conv(agg1, deg1, W1, b1[None, :], g1[None, :], be1[None, :])

    agg2, _ = _sc_conv2(h1, srcp, dstp, zrow, zdeg)
    return _tc_final(agg2, deg1, batch[None, :],
                     W2, b2[None, :], g2[None, :], be2[None, :],
                     Ws1, bs1[None, :], Ws2, bs2[None, :],
                     Wh1, bh1[None, :], Wh2, bh2[None, :],
                     Wh3, bh3[None, :])


# speedup vs baseline: 1.3061x; 1.3061x over previous
import functools

import jax
import jax.numpy as jnp
import numpy as np
from jax import lax
from jax.experimental import pallas as pl
from jax.experimental.pallas import tpu as pltpu
from jax.experimental.pallas import tpu_sc as plsc

N = 10000
E = 320000
D = 128
B = 100
NODES = 100

NC = 2
NS = 16
NW = NC * NS
CH = 88
T_FAST = 176
T_SLOW = 56
E_PAD = NS * (T_FAST + T_SLOW) * CH
N_PAD = 10112
RPT = N_PAD // NS

_ZROW = np.zeros((N_PAD, D), np.float32)
_ZDEG = np.zeros((N_PAD,), np.float32)

_mesh = plsc.VectorSubcoreMesh(core_axis_name="c", subcore_axis_name="s")


def _sc_body(with_deg, x_hbm, src_hbm, dst_hbm, zrow_hbm, zdeg_hbm,
             agg_hbm, deg_hbm,
             acc, srcs, rows0, rows1, db0, db1, hist,
             semg0, semg1, semd0, semd1, semsc):
    c = lax.axis_index("c")
    s = lax.axis_index("s")
    wid = c * NS + s
    nch = jnp.where(c == 0, T_FAST, T_SLOW)
    ebase = jnp.where(c == 0, s * (T_FAST * CH),
                      NS * (T_FAST * CH) + s * (T_SLOW * CH))

    @pl.when(c == 0)
    def _():
        pltpu.sync_copy(src_hbm.at[pl.ds(ebase, T_FAST * CH)], srcs)

    @pl.when(c == 1)
    def _():
        pltpu.sync_copy(src_hbm.at[pl.ds(ebase, T_SLOW * CH)],
                        srcs.at[pl.ds(0, T_SLOW * CH)])

    pltpu.sync_copy(zrow_hbm.at[pl.ds(s * RPT, RPT)], acc.at[pl.ds(s * RPT, RPT)])
    if with_deg:
        pltpu.sync_copy(zdeg_hbm, hist)
    plsc.subcore_barrier()

    rows = (rows0, rows1)
    semg = (semg0, semg1)
    dstb = (db0, db1)
    semd = (semd0, semd1)

    def gather(g, j2):
        return pltpu.make_async_copy(
            x_hbm.at[srcs.at[pl.ds(g * CH, CH)]], rows[j2], semg[j2])

    def dstload(g, j2):
        return pltpu.make_async_copy(
            dst_hbm.at[pl.ds(ebase + g * CH, CH)], dstb[j2], semd[j2])

    for b in range(2):
        gather(b, b).start()
        dstload(b, b).start()

    def step(k2, carry):
        for b in range(2):
            g = k2 * 2 + b
            gather(g, b).wait()
            dstload(g, b).wait()
            db = dstb[b]
            sc = pltpu.async_copy(rows[b], acc.at[db], semsc, add=True)
            if with_deg:
                for i in range(CH // 16):
                    dv = db[pl.ds(i * 16, 16)]
                    cnt, last = plsc.scan_count(dv)
                    old = plsc.load_gather(hist, [dv])
                    plsc.store_scatter(hist, [dv], old + cnt.astype(jnp.float32),
                                       mask=last)
            sc.wait()

            @pl.when(g + 2 < nch)
            def _():
                gather(g + 2, b).start()
                dstload(g + 2, b).start()
        return carry

    lax.fori_loop(0, nch // 2, step, 0)
    if with_deg:
        pltpu.sync_copy(hist, deg_hbm.at[pl.ds(wid * N_PAD, N_PAD)])
    plsc.subcore_barrier()

    out0 = c * N_PAD + s * RPT
    pltpu.sync_copy(acc.at[pl.ds(s * RPT, RPT)], agg_hbm.at[pl.ds(out0, RPT)])


def _make_sc(with_deg):
    return functools.partial(
        pl.kernel,
        out_type=[
            jax.ShapeDtypeStruct((NC * N_PAD, D), jnp.float32),
            jax.ShapeDtypeStruct((NW * N_PAD,), jnp.float32),
        ],
        mesh=_mesh,
        compiler_params=pltpu.CompilerParams(needs_layout_passes=False),
        scratch_types=[
            pltpu.VMEM_SHARED((N_PAD, D), jnp.float32),
            pltpu.VMEM((T_FAST * CH,), jnp.int32),
            pltpu.VMEM((CH, D), jnp.float32),
            pltpu.VMEM((CH, D), jnp.float32),
            pltpu.VMEM((CH,), jnp.int32),
            pltpu.VMEM((CH,), jnp.int32),
            pltpu.VMEM((N_PAD,), jnp.float32),
            pltpu.SemaphoreType.DMA,
            pltpu.SemaphoreType.DMA,
            pltpu.SemaphoreType.DMA,
            pltpu.SemaphoreType.DMA,
            pltpu.SemaphoreType.DMA,
        ],
    )(functools.partial(_sc_body, with_deg))


_sc_conv1 = _make_sc(True)
_sc_conv2 = _make_sc(False)


def _tc_conv_body(agg_ref, deg_ref, W_ref, b_ref, g_ref, be_ref, out_ref):
    a = agg_ref[0, :N, :] + agg_ref[1, :N, :]
    dg = jnp.sum(deg_ref[...], axis=1, keepdims=True)[:N, :]
    a = a * (1.0 / jnp.maximum(dg, 1.0))
    h = jnp.dot(a, W_ref[...], preferred_element_type=jnp.float32) + b_ref[...]
    m = jnp.mean(h, axis=0, keepdims=True)
    v = jnp.mean(h * h, axis=0, keepdims=True) - m * m
    hn = g_ref[...] * (h - m) * lax.rsqrt(v + 1e-5) + be_ref[...]
    out_ref[...] = jnp.maximum(hn, 0.0)


_tc_conv = pl.pallas_call(
    _tc_conv_body,
    out_shape=jax.ShapeDtypeStruct((N, D), jnp.float32),
    in_specs=[pl.BlockSpec(memory_space=pltpu.VMEM)] * 6,
    out_specs=pl.BlockSpec(memory_space=pltpu.VMEM),
)


def _tc_final_body(agg_ref, deg_ref, batch_ref,
                   W_ref, b_ref, g_ref, be_ref,
                   Ws1_ref, bs1_ref, Ws2_ref, bs2_ref,
                   Wh1_ref, bh1_ref, Wh2_ref, bh2_ref, Wh3_ref, bh3_ref,
                   out_ref):
    a = agg_ref[0, :N, :] + agg_ref[1, :N, :]
    dg = jnp.sum(deg_ref[...], axis=1, keepdims=True)[:N, :]
    a = a * (1.0 / jnp.maximum(dg, 1.0))
    h = jnp.dot(a, W_ref[...], preferred_element_type=jnp.float32) + b_ref[...]
    m = jnp.mean(h, axis=0, keepdims=True)
    v = jnp.mean(h * h, axis=0, keepdims=True) - m * m
    hn = g_ref[...] * (h - m) * lax.rsqrt(v + 1e-5) + be_ref[...]
    h2 = jnp.maximum(hn, 0.0)

    gids = lax.broadcasted_iota(jnp.int32, (B, N), 0)
    pmat = jnp.where(gids == batch_ref[...], 1.0 / NODES, 0.0)
    xg = jnp.dot(pmat, h2, preferred_element_type=jnp.float32)

    sh = jnp.maximum(xg, 0.0)
    sh = jnp.dot(sh, Ws1_ref[...], preferred_element_type=jnp.float32) + bs1_ref[...]
    sh = jnp.maximum(jnp.dot(sh, Ws2_ref[...], preferred_element_type=jnp.float32) + bs2_ref[...], 0.0)
    o = jnp.maximum(jnp.dot(sh, Wh1_ref[...], preferred_element_type=jnp.float32) + bh1_ref[...], 0.0)
    o = jnp.maximum(jnp.dot(o, Wh2_ref[...], preferred_element_type=jnp.float32) + bh2_ref[...], 0.0)
    out_ref[...] = jnp.dot(o, Wh3_ref[...], preferred_element_type=jnp.float32) + bh3_ref[...]


_tc_final = pl.pallas_call(
    _tc_final_body,
    out_shape=jax.ShapeDtypeStruct((B, 10), jnp.float32),
    in_specs=[pl.BlockSpec(memory_space=pltpu.VMEM)] * 17,
    out_specs=pl.BlockSpec(memory_space=pltpu.VMEM),
)


def kernel(x, edge_index, batch, W1, b1, g1, be1, W2, b2, g2, be2,
           Ws1, bs1, Ws2, bs2, Wh1, bh1, Wh2, bh2, Wh3, bh3):
    pad = E_PAD - E
    srcp = jnp.concatenate([edge_index[0], jnp.zeros((pad,), jnp.int32)])
    trash = N + (jnp.arange(pad, dtype=jnp.int32) % (N_PAD - N))
    dstp = jnp.concatenate([edge_index[1], trash])

    zrow = jnp.asarray(_ZROW)
    zdeg = jnp.asarray(_ZDEG)

    agg1, deg1 = _sc_conv1(x, srcp, dstp, zrow, zdeg)
    agg1 = agg1.reshape(NC, N_PAD, D)
    deg1 = deg1.reshape(NW, N_PAD).T
    h1 = _tc_conv(agg1, deg1, W1, b1[None, :], g1[None, :], be1[None, :])

    agg2, _ = _sc_conv2(h1, srcp, dstp, zrow, zdeg)
    agg2 = agg2.reshape(NC, N_PAD, D)
    return _tc_final(agg2, deg1, batch[None, :],
                     W2, b2[None, :], g2[None, :], be2[None, :],
                     Ws1, bs1[None, :], Ws2, bs2[None, :],
                     Wh1, bh1[None, :], Wh2, bh2[None, :],
                     Wh3, bh3[None, :])


# speedup vs baseline: 2.8678x; 2.1958x over previous
import functools

import jax
import jax.numpy as jnp
import numpy as np
from jax import lax
from jax.experimental import pallas as pl
from jax.experimental.pallas import tpu as pltpu
from jax.experimental.pallas import tpu_sc as plsc

N = 10000
E = 320000
D = 128
B = 100
NODES = 100

NC = 2
NS = 16
NW = NC * NS
CH = 80
T_FAST = 188
T_SLOW = 62
E_PAD = NS * (T_FAST + T_SLOW) * CH
N_PAD = 10112
RPT = N_PAD // NS

_ZROW = np.zeros((N_PAD, D), np.float32)
_ZDEG = np.zeros((N_PAD,), np.float32)

_mesh = plsc.VectorSubcoreMesh(core_axis_name="c", subcore_axis_name="s")


def _sc_body(with_deg, x_hbm, src_hbm, dst_hbm, zrow_hbm, zdeg_hbm,
             agg_hbm, deg_hbm,
             acc, srcs, rows0, rows1, db0, db1, hist,
             semg0, semg1, semd0, semd1, semsc):
    c = lax.axis_index("c")
    s = lax.axis_index("s")
    wid = c * NS + s
    nch = jnp.where(c == 0, T_FAST, T_SLOW)
    ebase = jnp.where(c == 0, s * (T_FAST * CH),
                      NS * (T_FAST * CH) + s * (T_SLOW * CH))

    @pl.when(c == 0)
    def _():
        pltpu.sync_copy(src_hbm.at[pl.ds(ebase, T_FAST * CH)], srcs)

    @pl.when(c == 1)
    def _():
        pltpu.sync_copy(src_hbm.at[pl.ds(ebase, T_SLOW * CH)],
                        srcs.at[pl.ds(0, T_SLOW * CH)])

    pltpu.sync_copy(zrow_hbm.at[pl.ds(s * RPT, RPT)], acc.at[pl.ds(s * RPT, RPT)])
    if with_deg:
        pltpu.sync_copy(zdeg_hbm, hist)
    plsc.subcore_barrier()

    rows = (rows0, rows1)
    semg = (semg0, semg1)
    dstb = (db0, db1)
    semd = (semd0, semd1)

    def gather(g, j2):
        return pltpu.make_async_copy(
            x_hbm.at[srcs.at[pl.ds(g * CH, CH)]], rows[j2], semg[j2])

    def dstload(g, j2):
        return pltpu.make_async_copy(
            dst_hbm.at[pl.ds(ebase + g * CH, CH)], dstb[j2], semd[j2])

    for b in range(2):
        gather(b, b).start()
        dstload(b, b).start()

    def step(k2, carry):
        for b in range(2):
            g = k2 * 2 + b
            gather(g, b).wait()
            dstload(g, b).wait()
            db = dstb[b]
            sc = pltpu.async_copy(rows[b], acc.at[db], semsc, add=True)
            if with_deg:
                for i in range(CH // 16):
                    dv = db[pl.ds(i * 16, 16)]
                    cnt, last = plsc.scan_count(dv)
                    old = plsc.load_gather(hist, [dv])
                    plsc.store_scatter(hist, [dv], old + cnt.astype(jnp.float32),
                                       mask=last)
            sc.wait()

            @pl.when(g + 2 < nch)
            def _():
                gather(g + 2, b).start()
                dstload(g + 2, b).start()
        return carry

    lax.fori_loop(0, nch // 2, step, 0)
    if with_deg:
        pltpu.sync_copy(hist, deg_hbm.at[pl.ds(wid * N_PAD, N_PAD)])
    plsc.subcore_barrier()

    out0 = c * N_PAD + s * RPT
    pltpu.sync_copy(acc.at[pl.ds(s * RPT, RPT)], agg_hbm.at[pl.ds(out0, RPT)])


def _make_sc(with_deg):
    return functools.partial(
        pl.kernel,
        out_type=[
            jax.ShapeDtypeStruct((NC * N_PAD, D), jnp.float32),
            jax.ShapeDtypeStruct((NW * N_PAD,), jnp.float32),
        ],
        mesh=_mesh,
        compiler_params=pltpu.CompilerParams(needs_layout_passes=False),
        scratch_types=[
            pltpu.VMEM_SHARED((N_PAD, D), jnp.float32),
            pltpu.VMEM((T_FAST * CH,), jnp.int32),
            pltpu.VMEM((CH, D), jnp.float32),
            pltpu.VMEM((CH, D), jnp.float32),
            pltpu.VMEM((CH,), jnp.int32),
            pltpu.VMEM((CH,), jnp.int32),
            pltpu.VMEM((N_PAD,), jnp.float32),
            pltpu.SemaphoreType.DMA,
            pltpu.SemaphoreType.DMA,
            pltpu.SemaphoreType.DMA,
            pltpu.SemaphoreType.DMA,
            pltpu.SemaphoreType.DMA,
        ],
    )(functools.partial(_sc_body, with_deg))


_sc_conv1 = _make_sc(True)
_sc_conv2 = _make_sc(False)


def _tc_conv_body(agg_ref, deg_ref, W_ref, b_ref, g_ref, be_ref, out_ref):
    a = agg_ref[0, :N, :] + agg_ref[1, :N, :]
    dg = jnp.sum(deg_ref[...], axis=1, keepdims=True)[:N, :]
    a = a * (1.0 / jnp.maximum(dg, 1.0))
    h = jnp.dot(a, W_ref[...], preferred_element_type=jnp.float32) + b_ref[...]
    m = jnp.mean(h, axis=0, keepdims=True)
    v = jnp.mean(h * h, axis=0, keepdims=True) - m * m
    hn = g_ref[...] * (h - m) * lax.rsqrt(v + 1e-5) + be_ref[...]
    out_ref[...] = jnp.maximum(hn, 0.0)


_tc_conv = pl.pallas_call(
    _tc_conv_body,
    out_shape=jax.ShapeDtypeStruct((N, D), jnp.float32),
    in_specs=[pl.BlockSpec(memory_space=pltpu.VMEM)] * 6,
    out_specs=pl.BlockSpec(memory_space=pltpu.VMEM),
)


def _tc_final_body(agg_ref, deg_ref, batch_ref,
                   W_ref, b_ref, g_ref, be_ref,
                   Ws1_ref, bs1_ref, Ws2_ref, bs2_ref,
                   Wh1_ref, bh1_ref, Wh2_ref, bh2_ref, Wh3_ref, bh3_ref,
                   out_ref):
    a = agg_ref[0, :N, :] + agg_ref[1, :N, :]
    dg = jnp.sum(deg_ref[...], axis=1, keepdims=True)[:N, :]
    a = a * (1.0 / jnp.maximum(dg, 1.0))
    h = jnp.dot(a, W_ref[...], preferred_element_type=jnp.float32) + b_ref[...]
    m = jnp.mean(h, axis=0, keepdims=True)
    v = jnp.mean(h * h, axis=0, keepdims=True) - m * m
    hn = g_ref[...] * (h - m) * lax.rsqrt(v + 1e-5) + be_ref[...]
    h2 = jnp.maximum(hn, 0.0)

    gids = lax.broadcasted_iota(jnp.int32, (B, N), 0)
    pmat = jnp.where(gids == batch_ref[...], 1.0 / NODES, 0.0)
    xg = jnp.dot(pmat, h2, preferred_element_type=jnp.float32)

    sh = jnp.maximum(xg, 0.0)
    sh = jnp.dot(sh, Ws1_ref[...], preferred_element_type=jnp.float32) + bs1_ref[...]
    sh = jnp.maximum(jnp.dot(sh, Ws2_ref[...], preferred_element_type=jnp.float32) + bs2_ref[...], 0.0)
    o = jnp.maximum(jnp.dot(sh, Wh1_ref[...], preferred_element_type=jnp.float32) + bh1_ref[...], 0.0)
    o = jnp.maximum(jnp.dot(o, Wh2_ref[...], preferred_element_type=jnp.float32) + bh2_ref[...], 0.0)
    out_ref[...] = jnp.dot(o, Wh3_ref[...], preferred_element_type=jnp.float32) + bh3_ref[...]


_tc_final = pl.pallas_call(
    _tc_final_body,
    out_shape=jax.ShapeDtypeStruct((B, 10), jnp.float32),
    in_specs=[pl.BlockSpec(memory_space=pltpu.VMEM)] * 17,
    out_specs=pl.BlockSpec(memory_space=pltpu.VMEM),
)


def kernel(x, edge_index, batch, W1, b1, g1, be1, W2, b2, g2, be2,
           Ws1, bs1, Ws2, bs2, Wh1, bh1, Wh2, bh2, Wh3, bh3):
    pad = E_PAD - E
    srcp = jnp.concatenate([edge_index[0], jnp.zeros((pad,), jnp.int32)])
    trash = N + (jnp.arange(pad, dtype=jnp.int32) % (N_PAD - N))
    dstp = jnp.concatenate([edge_index[1], trash])

    zrow = jnp.asarray(_ZROW)
    zdeg = jnp.asarray(_ZDEG)

    agg1, deg1 = _sc_conv1(x, srcp, dstp, zrow, zdeg)
    agg1 = agg1.reshape(NC, N_PAD, D)
    deg1 = deg1.reshape(NW, N_PAD).T
    h1 = _tc_conv(agg1, deg1, W1, b1[None, :], g1[None, :], be1[None, :])

    agg2, _ = _sc_conv2(h1, srcp, dstp, zrow, zdeg)
    agg2 = agg2.reshape(NC, N_PAD, D)
    return _tc_final(agg2, deg1, batch[None, :],
                     W2, b2[None, :], g2[None, :], be2[None, :],
                     Ws1, bs1[None, :], Ws2, bs2[None, :],
                     Wh1, bh1[None, :], Wh2, bh2[None, :],
                     Wh3, bh3[None, :])


# speedup vs baseline: 3.5928x; 1.2528x over previous
import functools

import jax
import jax.numpy as jnp
import numpy as np
from jax import lax
from jax.experimental import pallas as pl
from jax.experimental.pallas import tpu as pltpu
from jax.experimental.pallas import tpu_sc as plsc

N = 10000
E = 320000
D = 128
B = 100
NODES = 100

NC = 2
NS = 16
NW = NC * NS
CH = 80
T_FAST = 136
T_SLOW = 114
E_PAD = NS * (T_FAST + T_SLOW) * CH
N_PAD = 10112
RPT = N_PAD // NS

_ZROW = np.zeros((N_PAD, D), np.float32)
_ZDEG = np.zeros((N_PAD,), np.float32)

_mesh = plsc.VectorSubcoreMesh(core_axis_name="c", subcore_axis_name="s")


def _sc_body(with_deg, x_hbm, src_hbm, dst_hbm, zrow_hbm, zdeg_hbm,
             agg_hbm, deg_hbm,
             acc, srcs, rows0, rows1, db0, db1, hist,
             semg0, semg1, semd0, semd1, semsc):
    c = lax.axis_index("c")
    s = lax.axis_index("s")
    wid = c * NS + s
    nch = jnp.where(c == 0, T_FAST, T_SLOW)
    ebase = jnp.where(c == 0, s * (T_FAST * CH),
                      NS * (T_FAST * CH) + s * (T_SLOW * CH))

    @pl.when(c == 0)
    def _():
        pltpu.sync_copy(src_hbm.at[pl.ds(ebase, T_FAST * CH)], srcs)

    @pl.when(c == 1)
    def _():
        pltpu.sync_copy(src_hbm.at[pl.ds(ebase, T_SLOW * CH)],
                        srcs.at[pl.ds(0, T_SLOW * CH)])

    pltpu.sync_copy(zrow_hbm.at[pl.ds(s * RPT, RPT)], acc.at[pl.ds(s * RPT, RPT)])
    if with_deg:
        pltpu.sync_copy(zdeg_hbm, hist)
    plsc.subcore_barrier()

    rows = (rows0, rows1)
    semg = (semg0, semg1)
    dstb = (db0, db1)
    semd = (semd0, semd1)

    def gather(g, j2):
        return pltpu.make_async_copy(
            x_hbm.at[srcs.at[pl.ds(g * CH, CH)]], rows[j2], semg[j2])

    def dstload(g, j2):
        return pltpu.make_async_copy(
            dst_hbm.at[pl.ds(ebase + g * CH, CH)], dstb[j2], semd[j2])

    for b in range(2):
        gather(b, b).start()
        dstload(b, b).start()

    def step(k2, carry):
        for b in range(2):
            g = k2 * 2 + b
            gather(g, b).wait()
            dstload(g, b).wait()
            db = dstb[b]
            sc = pltpu.async_copy(rows[b], acc.at[db], semsc, add=True)
            if with_deg:
                for i in range(CH // 16):
                    dv = db[pl.ds(i * 16, 16)]
                    cnt, last = plsc.scan_count(dv)
                    old = plsc.load_gather(hist, [dv])
                    plsc.store_scatter(hist, [dv], old + cnt.astype(jnp.float32),
                                       mask=last)
            sc.wait()

            @pl.when(g + 2 < nch)
            def _():
                gather(g + 2, b).start()
                dstload(g + 2, b).start()
        return carry

    lax.fori_loop(0, nch // 2, step, 0)
    if with_deg:
        pltpu.sync_copy(hist, deg_hbm.at[pl.ds(wid * N_PAD, N_PAD)])
    plsc.subcore_barrier()

    out0 = c * N_PAD + s * RPT
    pltpu.sync_copy(acc.at[pl.ds(s * RPT, RPT)], agg_hbm.at[pl.ds(out0, RPT)])


def _make_sc(with_deg):
    return functools.partial(
        pl.kernel,
        out_type=[
            jax.ShapeDtypeStruct((NC * N_PAD, D), jnp.float32),
            jax.ShapeDtypeStruct((NW * N_PAD,), jnp.float32),
        ],
        mesh=_mesh,
        compiler_params=pltpu.CompilerParams(needs_layout_passes=False),
        scratch_types=[
            pltpu.VMEM_SHARED((N_PAD, D), jnp.float32),
            pltpu.VMEM((T_FAST * CH,), jnp.int32),
            pltpu.VMEM((CH, D), jnp.float32),
            pltpu.VMEM((CH, D), jnp.float32),
            pltpu.VMEM((CH,), jnp.int32),
            pltpu.VMEM((CH,), jnp.int32),
            pltpu.VMEM((N_PAD,), jnp.float32),
            pltpu.SemaphoreType.DMA,
            pltpu.SemaphoreType.DMA,
            pltpu.SemaphoreType.DMA,
            pltpu.SemaphoreType.DMA,
            pltpu.SemaphoreType.DMA,
        ],
    )(functools.partial(_sc_body, with_deg))


_sc_conv1 = _make_sc(True)
_sc_conv2 = _make_sc(False)


def _tc_conv_body(agg_ref, deg_ref, W_ref, b_ref, g_ref, be_ref, out_ref):
    a = agg_ref[0, :N, :] + agg_ref[1, :N, :]
    dg = jnp.sum(deg_ref[...], axis=1, keepdims=True)[:N, :]
    a = a * (1.0 / jnp.maximum(dg, 1.0))
    h = jnp.dot(a, W_ref[...], preferred_element_type=jnp.float32) + b_ref[...]
    m = jnp.mean(h, axis=0, keepdims=True)
    v = jnp.mean(h * h, axis=0, keepdims=True) - m * m
    hn = g_ref[...] * (h - m) * lax.rsqrt(v + 1e-5) + be_ref[...]
    out_ref[...] = jnp.maximum(hn, 0.0)


_tc_conv = pl.pallas_call(
    _tc_conv_body,
    out_shape=jax.ShapeDtypeStruct((N, D), jnp.float32),
    in_specs=[pl.BlockSpec(memory_space=pltpu.VMEM)] * 6,
    out_specs=pl.BlockSpec(memory_space=pltpu.VMEM),
)


def _tc_final_body(agg_ref, deg_ref, batch_ref,
                   W_ref, b_ref, g_ref, be_ref,
                   Ws1_ref, bs1_ref, Ws2_ref, bs2_ref,
                   Wh1_ref, bh1_ref, Wh2_ref, bh2_ref, Wh3_ref, bh3_ref,
                   out_ref):
    a = agg_ref[0, :N, :] + agg_ref[1, :N, :]
    dg = jnp.sum(deg_ref[...], axis=1, keepdims=True)[:N, :]
    a = a * (1.0 / jnp.maximum(dg, 1.0))
    h = jnp.dot(a, W_ref[...], preferred_element_type=jnp.float32) + b_ref[...]
    m = jnp.mean(h, axis=0, keepdims=True)
    v = jnp.mean(h * h, axis=0, keepdims=True) - m * m
    hn = g_ref[...] * (h - m) * lax.rsqrt(v + 1e-5) + be_ref[...]
    h2 = jnp.maximum(hn, 0.0)

    gids = lax.broadcasted_iota(jnp.int32, (B, N), 0)
    pmat = jnp.where(gids == batch_ref[...], 1.0 / NODES, 0.0)
    xg = jnp.dot(pmat, h2, preferred_element_type=jnp.float32)

    sh = jnp.maximum(xg, 0.0)
    sh = jnp.dot(sh, Ws1_ref[...], preferred_element_type=jnp.float32) + bs1_ref[...]
    sh = jnp.maximum(jnp.dot(sh, Ws2_ref[...], preferred_element_type=jnp.float32) + bs2_ref[...], 0.0)
    o = jnp.maximum(jnp.dot(sh, Wh1_ref[...], preferred_element_type=jnp.float32) + bh1_ref[...], 0.0)
    o = jnp.maximum(jnp.dot(o, Wh2_ref[...], preferred_element_type=jnp.float32) + bh2_ref[...], 0.0)
    out_ref[...] = jnp.dot(o, Wh3_ref[...], preferred_element_type=jnp.float32) + bh3_ref[...]


_tc_final = pl.pallas_call(
    _tc_final_body,
    out_shape=jax.ShapeDtypeStruct((B, 10), jnp.float32),
    in_specs=[pl.BlockSpec(memory_space=pltpu.VMEM)] * 17,
    out_specs=pl.BlockSpec(memory_space=pltpu.VMEM),
)


def kernel(x, edge_index, batch, W1, b1, g1, be1, W2, b2, g2, be2,
           Ws1, bs1, Ws2, bs2, Wh1, bh1, Wh2, bh2, Wh3, bh3):
    pad = E_PAD - E
    srcp = jnp.concatenate([edge_index[0], jnp.zeros((pad,), jnp.int32)])
    trash = N + (jnp.arange(pad, dtype=jnp.int32) % (N_PAD - N))
    dstp = jnp.concatenate([edge_index[1], trash])

    zrow = jnp.asarray(_ZROW)
    zdeg = jnp.asarray(_ZDEG)

    agg1, deg1 = _sc_conv1(x, srcp, dstp, zrow, zdeg)
    agg1 = agg1.reshape(NC, N_PAD, D)
    deg1 = deg1.reshape(NW, N_PAD).T
    h1 = _tc_conv(agg1, deg1, W1, b1[None, :], g1[None, :], be1[None, :])

    agg2, _ = _sc_conv2(h1, srcp, dstp, zrow, zdeg)
    agg2 = agg2.reshape(NC, N_PAD, D)
    return _tc_final(agg2, deg1, batch[None, :],
                     W2, b2[None, :], g2[None, :], be2[None, :],
                     Ws1, bs1[None, :], Ws2, bs2[None, :],
                     Wh1, bh1[None, :], Wh2, bh2[None, :],
                     Wh3, bh3[None, :])


# speedup vs baseline: 3.7395x; 1.0409x over previous
import functools

import jax
import jax.numpy as jnp
import numpy as np
from jax import lax
from jax.experimental import pallas as pl
from jax.experimental.pallas import tpu as pltpu
from jax.experimental.pallas import tpu_sc as plsc

N = 10000
E = 320000
D = 128
B = 100
NODES = 100

NC = 2
NS = 16
NW = NC * NS
CH = 80
T_FAST = 128
T_SLOW = 122
E_PAD = NS * (T_FAST + T_SLOW) * CH
N_PAD = 10112
RPT = N_PAD // NS

_ZROW = np.zeros((N_PAD, D), np.float32)
_ZDEG = np.zeros((N_PAD,), np.float32)

_mesh = plsc.VectorSubcoreMesh(core_axis_name="c", subcore_axis_name="s")


def _sc_body(with_deg, x_hbm, src_hbm, dst_hbm, zrow_hbm, zdeg_hbm,
             agg_hbm, deg_hbm,
             acc, srcs, rows0, rows1, db0, db1, hist,
             semg0, semg1, semd0, semd1, semsc):
    c = lax.axis_index("c")
    s = lax.axis_index("s")
    wid = c * NS + s
    nch = jnp.where(c == 0, T_FAST, T_SLOW)
    ebase = jnp.where(c == 0, s * (T_FAST * CH),
                      NS * (T_FAST * CH) + s * (T_SLOW * CH))

    @pl.when(c == 0)
    def _():
        pltpu.sync_copy(src_hbm.at[pl.ds(ebase, T_FAST * CH)], srcs)

    @pl.when(c == 1)
    def _():
        pltpu.sync_copy(src_hbm.at[pl.ds(ebase, T_SLOW * CH)],
                        srcs.at[pl.ds(0, T_SLOW * CH)])

    pltpu.sync_copy(zrow_hbm.at[pl.ds(s * RPT, RPT)], acc.at[pl.ds(s * RPT, RPT)])
    if with_deg:
        pltpu.sync_copy(zdeg_hbm, hist)
    plsc.subcore_barrier()

    rows = (rows0, rows1)
    semg = (semg0, semg1)
    dstb = (db0, db1)
    semd = (semd0, semd1)

    def gather(g, j2):
        return pltpu.make_async_copy(
            x_hbm.at[srcs.at[pl.ds(g * CH, CH)]], rows[j2], semg[j2])

    def dstload(g, j2):
        return pltpu.make_async_copy(
            dst_hbm.at[pl.ds(ebase + g * CH, CH)], dstb[j2], semd[j2])

    for b in range(2):
        gather(b, b).start()
        dstload(b, b).start()

    def step(k2, carry):
        for b in range(2):
            g = k2 * 2 + b
            gather(g, b).wait()
            dstload(g, b).wait()
            db = dstb[b]
            sc = pltpu.async_copy(rows[b], acc.at[db], semsc, add=True)
            if with_deg:
                for i in range(CH // 16):
                    dv = db[pl.ds(i * 16, 16)]
                    cnt, last = plsc.scan_count(dv)
                    old = plsc.load_gather(hist, [dv])
                    plsc.store_scatter(hist, [dv], old + cnt.astype(jnp.float32),
                                       mask=last)
            sc.wait()

            @pl.when(g + 2 < nch)
            def _():
                gather(g + 2, b).start()
                dstload(g + 2, b).start()
        return carry

    lax.fori_loop(0, nch // 2, step, 0)
    if with_deg:
        pltpu.sync_copy(hist, deg_hbm.at[pl.ds(wid * N_PAD, N_PAD)])
    plsc.subcore_barrier()

    out0 = c * N_PAD + s * RPT
    pltpu.sync_copy(acc.at[pl.ds(s * RPT, RPT)], agg_hbm.at[pl.ds(out0, RPT)])


def _make_sc(with_deg):
    return functools.partial(
        pl.kernel,
        out_type=[
            jax.ShapeDtypeStruct((NC * N_PAD, D), jnp.float32),
            jax.ShapeDtypeStruct((NW * N_PAD,), jnp.float32),
        ],
        mesh=_mesh,
        compiler_params=pltpu.CompilerParams(needs_layout_passes=False),
        scratch_types=[
            pltpu.VMEM_SHARED((N_PAD, D), jnp.float32),
            pltpu.VMEM((T_FAST * CH,), jnp.int32),
            pltpu.VMEM((CH, D), jnp.float32),
            pltpu.VMEM((CH, D), jnp.float32),
            pltpu.VMEM((CH,), jnp.int32),
            pltpu.VMEM((CH,), jnp.int32),
            pltpu.VMEM((N_PAD,), jnp.float32),
            pltpu.SemaphoreType.DMA,
            pltpu.SemaphoreType.DMA,
            pltpu.SemaphoreType.DMA,
            pltpu.SemaphoreType.DMA,
            pltpu.SemaphoreType.DMA,
        ],
    )(functools.partial(_sc_body, with_deg))


_sc_conv1 = _make_sc(True)
_sc_conv2 = _make_sc(False)


def _tc_conv_body(agg_ref, deg_ref, W_ref, b_ref, g_ref, be_ref, out_ref):
    a = agg_ref[0, :N, :] + agg_ref[1, :N, :]
    dg = jnp.sum(deg_ref[...], axis=1, keepdims=True)[:N, :]
    a = a * (1.0 / jnp.maximum(dg, 1.0))
    h = jnp.dot(a, W_ref[...], preferred_element_type=jnp.float32) + b_ref[...]
    m = jnp.mean(h, axis=0, keepdims=True)
    v = jnp.mean(h * h, axis=0, keepdims=True) - m * m
    hn = g_ref[...] * (h - m) * lax.rsqrt(v + 1e-5) + be_ref[...]
    out_ref[...] = jnp.maximum(hn, 0.0)


_tc_conv = pl.pallas_call(
    _tc_conv_body,
    out_shape=jax.ShapeDtypeStruct((N, D), jnp.float32),
    in_specs=[pl.BlockSpec(memory_space=pltpu.VMEM)] * 6,
    out_specs=pl.BlockSpec(memory_space=pltpu.VMEM),
)


def _tc_final_body(agg_ref, deg_ref, batch_ref,
                   W_ref, b_ref, g_ref, be_ref,
                   Ws1_ref, bs1_ref, Ws2_ref, bs2_ref,
                   Wh1_ref, bh1_ref, Wh2_ref, bh2_ref, Wh3_ref, bh3_ref,
                   out_ref):
    a = agg_ref[0, :N, :] + agg_ref[1, :N, :]
    dg = jnp.sum(deg_ref[...], axis=1, keepdims=True)[:N, :]
    a = a * (1.0 / jnp.maximum(dg, 1.0))
    h = jnp.dot(a, W_ref[...], preferred_element_type=jnp.float32) + b_ref[...]
    m = jnp.mean(h, axis=0, keepdims=True)
    v = jnp.mean(h * h, axis=0, keepdims=True) - m * m
    hn = g_ref[...] * (h - m) * lax.rsqrt(v + 1e-5) + be_ref[...]
    h2 = jnp.maximum(hn, 0.0)

    gids = lax.broadcasted_iota(jnp.int32, (B, N), 0)
    pmat = jnp.where(gids == batch_ref[...], 1.0 / NODES, 0.0)
    xg = jnp.dot(pmat, h2, preferred_element_type=jnp.float32)

    sh = jnp.maximum(xg, 0.0)
    sh = jnp.dot(sh, Ws1_ref[...], preferred_element_type=jnp.float32) + bs1_ref[...]
    sh = jnp.maximum(jnp.dot(sh, Ws2_ref[...], preferred_element_type=jnp.float32) + bs2_ref[...], 0.0)
    o = jnp.maximum(jnp.dot(sh, Wh1_ref[...], preferred_element_type=jnp.float32) + bh1_ref[...], 0.0)
    o = jnp.maximum(jnp.dot(o, Wh2_ref[...], preferred_element_type=jnp.float32) + bh2_ref[...], 0.0)
    out_ref[...] = jnp.dot(o, Wh3_ref[...], preferred_element_type=jnp.float32) + bh3_ref[...]


_tc_final = pl.pallas_call(
    _tc_final_body,
    out_shape=jax.ShapeDtypeStruct((B, 10), jnp.float32),
    in_specs=[pl.BlockSpec(memory_space=pltpu.VMEM)] * 17,
    out_specs=pl.BlockSpec(memory_space=pltpu.VMEM),
)


def kernel(x, edge_index, batch, W1, b1, g1, be1, W2, b2, g2, be2,
           Ws1, bs1, Ws2, bs2, Wh1, bh1, Wh2, bh2, Wh3, bh3):
    pad = E_PAD - E
    srcp = jnp.concatenate([edge_index[0], jnp.zeros((pad,), jnp.int32)])
    trash = N + (jnp.arange(pad, dtype=jnp.int32) % (N_PAD - N))
    dstp = jnp.concatenate([edge_index[1], trash])

    zrow = jnp.asarray(_ZROW)
    zdeg = jnp.asarray(_ZDEG)

    agg1, deg1 = _sc_conv1(x, srcp, dstp, zrow, zdeg)
    agg1 = agg1.reshape(NC, N_PAD, D)
    deg1 = deg1.reshape(NW, N_PAD).T
    h1 = _tc_conv(agg1, deg1, W1, b1[None, :], g1[None, :], be1[None, :])

    agg2, _ = _sc_conv2(h1, srcp, dstp, zrow, zdeg)
    agg2 = agg2.reshape(NC, N_PAD, D)
    return _tc_final(agg2, deg1, batch[None, :],
                     W2, b2[None, :], g2[None, :], be2[None, :],
                     Ws1, bs1[None, :], Ws2, bs2[None, :],
                     Wh1, bh1[None, :], Wh2, bh2[None, :],
                     Wh3, bh3[None, :])
